# Initial kernel scaffold; baseline (speedup 1.0000x reference)
import jax
import jax.numpy as jnp
from jax.experimental import pallas as pl


def kernel(x, edge_index, Ws1, Wn1, b1, Ws2, Wn2, b2, Ws3, Wn3, b3):
    raise NotImplementedError("write your pallas kernel here")


# speedup vs baseline: 6.4020x; 6.4020x over previous
import jax
import jax.numpy as jnp
from jax import lax
from jax.experimental import pallas as pl
from jax.experimental.pallas import tpu as pltpu
from jax.experimental.pallas import tpu_sc as plsc

_N = 10000
_E = 160000
_NPAD = 10240
_EPAD = 163840
_NT = 16
_NCH = 160
_NCH2 = 80
_CW = 64
_RPT = _NPAD // _NT
_DEGW = 16
_ROWB = 1000

_MESH = dict(core_axis_name="c", subcore_axis_name="s")


def _edge_pass(hn0, hn1, src_i, dst_i, W):
  outs = [jax.ShapeDtypeStruct((_NPAD, W), jnp.float32),
          jax.ShapeDtypeStruct((_NPAD, W), jnp.float32)]

  scratch = [
      pltpu.VMEM((_NCH2, _CW), jnp.int32),
      pltpu.VMEM((_NCH2, _CW), jnp.int32),
      pltpu.VMEM((_CW, W), jnp.float32),
      pltpu.VMEM((_CW, W), jnp.float32),
      pltpu.VMEM((16, W), jnp.float32),
      pltpu.VMEM_SHARED((_NPAD, W), jnp.float32),
      pltpu.SemaphoreType.DMA,
      pltpu.SemaphoreType.DMA,
  ]

  def body(hn0_ref, hn1_ref, src_ref, dst_ref, agg0_o, agg1_o,
           src_v, dst_v, rows0, rows1, zbuf, aggm, sem0, sem1):
    c = lax.axis_index("c")
    s = lax.axis_index("s")

    z16 = jnp.zeros((16,), jnp.float32)
    for r in range(16):
      for k in range(W // 16):
        zbuf[r, pl.ds(k * 16, 16)] = z16

    def zero_agg(i, acc):
      pltpu.sync_copy(zbuf, aggm.at[pl.ds(s * _RPT + i * 16, 16)])
      return acc
    lax.fori_loop(0, _RPT // 16, zero_agg, 0)

    plsc.subcore_barrier()

    def run(hn_ref):
      def chunk_pair(j2, acc):
        a = j2 * 2
        ca = pltpu.async_copy(hn_ref.at[src_v.at[a]], rows0, sem0)
        cb = pltpu.async_copy(hn_ref.at[src_v.at[a + 1]], rows1, sem1)
        ca.wait()
        pltpu.sync_copy(rows0, aggm.at[dst_v.at[a]], add=True)
        cb.wait()
        pltpu.sync_copy(rows1, aggm.at[dst_v.at[a + 1]], add=True)
        return acc
      for h in range(_NCH // _NCH2):
        pltpu.sync_copy(src_ref.at[s, pl.ds(h * _NCH2, _NCH2)], src_v)
        pltpu.sync_copy(dst_ref.at[s, pl.ds(h * _NCH2, _NCH2)], dst_v)
        lax.fori_loop(0, _NCH2 // 2, chunk_pair, 0)

    pl.when(c == 0)(lambda: run(hn0_ref))
    pl.when(c == 1)(lambda: run(hn1_ref))
    plsc.subcore_barrier()

    rows_out = pl.ds(s * _RPT, _RPT)
    pl.when(c == 0)(lambda: pltpu.sync_copy(aggm.at[rows_out],
                                            agg0_o.at[rows_out]))
    pl.when(c == 1)(lambda: pltpu.sync_copy(aggm.at[rows_out],
                                            agg1_o.at[rows_out]))

  mesh = plsc.VectorSubcoreMesh(**_MESH)
  call = pl.kernel(body, out_type=outs, mesh=mesh, scratch_types=scratch)
  return call(hn0, hn1, src_i, dst_i)


def _edge_pass_split(hn, src_i4, dst_i4, W):
  outs = jax.ShapeDtypeStruct((2, _NPAD, W), jnp.float32)

  scratch = [
      pltpu.VMEM((_NCH2, _CW), jnp.int32),
      pltpu.VMEM((_NCH2, _CW), jnp.int32),
      pltpu.VMEM((_CW, W), jnp.float32),
      pltpu.VMEM((_CW, W), jnp.float32),
      pltpu.VMEM((16, W), jnp.float32),
      pltpu.VMEM_SHARED((_NPAD, W), jnp.float32),
      pltpu.SemaphoreType.DMA,
      pltpu.SemaphoreType.DMA,
  ]

  def body(hn_ref, src_ref, dst_ref, p_o,
           src_v, dst_v, rows0, rows1, zbuf, aggm, sem0, sem1):
    c = lax.axis_index("c")
    s = lax.axis_index("s")

    z16 = jnp.zeros((16,), jnp.float32)
    for r in range(16):
      for k in range(W // 16):
        zbuf[r, pl.ds(k * 16, 16)] = z16

    def zero_agg(i, acc):
      pltpu.sync_copy(zbuf, aggm.at[pl.ds(s * _RPT + i * 16, 16)])
      return acc
    lax.fori_loop(0, _RPT // 16, zero_agg, 0)

    pltpu.sync_copy(src_ref.at[c, s], src_v)
    pltpu.sync_copy(dst_ref.at[c, s], dst_v)
    plsc.subcore_barrier()

    def chunk_pair(j2, acc):
      a = j2 * 2
      ca = pltpu.async_copy(hn_ref.at[src_v.at[a]], rows0, sem0)
      cb = pltpu.async_copy(hn_ref.at[src_v.at[a + 1]], rows1, sem1)
      ca.wait()
      pltpu.sync_copy(rows0, aggm.at[dst_v.at[a]], add=True)
      cb.wait()
      pltpu.sync_copy(rows1, aggm.at[dst_v.at[a + 1]], add=True)
      return acc
    lax.fori_loop(0, _NCH2 // 2, chunk_pair, 0)
    plsc.subcore_barrier()

    rows_out = pl.ds(s * _RPT, _RPT)
    pltpu.sync_copy(aggm.at[rows_out], p_o.at[c, rows_out])

  mesh = plsc.VectorSubcoreMesh(**_MESH)
  call = pl.kernel(body, out_type=outs, mesh=mesh, scratch_types=scratch)
  return call(hn, src_i4, dst_i4)


def _deg_pass(dst_i4):
  out = jax.ShapeDtypeStruct((2, _NPAD), jnp.float32)

  scratch = [
      pltpu.VMEM((_NCH2, _CW), jnp.int32),
      pltpu.VMEM((8 * _NPAD,), jnp.float32),
      pltpu.VMEM((_NPAD,), jnp.float32),
      pltpu.VMEM((_RPT,), jnp.float32),
      pltpu.VMEM_SHARED((_NT, _NPAD), jnp.float32),
  ]

  def body(dst_ref, deg_o, dst_v, hist, tsum, rbuf, stage):
    c = lax.axis_index("c")
    s = lax.axis_index("s")
    pltpu.sync_copy(dst_ref.at[c, s], dst_v)

    z16 = jnp.zeros((16,), jnp.float32)

    def zrow(i, acc):
      for r in range(8):
        hist[pl.ds(r * _NPAD + i * 16, 16)] = z16
      return acc
    lax.fori_loop(0, _NPAD // 16, zrow, 0)

    lanes = lax.iota(jnp.int32, 16)
    base = (lanes & 7) * _NPAD
    lo = lanes < 8
    hi = lanes >= 8
    ones = jnp.ones((16,), jnp.float32)

    def count(j, acc):
      for k in range(_CW // 16):
        d = dst_v[j, pl.ds(k * 16, 16)]
        plsc.addupdate_scatter(hist, [base + d], ones, mask=lo)
        plsc.addupdate_scatter(hist, [base + d], ones, mask=hi)
      return acc
    lax.fori_loop(0, _NCH2, count, 0)

    def sumrows(i, acc):
      t = hist[pl.ds(i * 16, 16)]
      for r in range(1, 8):
        t = t + hist[pl.ds(r * _NPAD + i * 16, 16)]
      tsum[pl.ds(i * 16, 16)] = t
      return acc
    lax.fori_loop(0, _NPAD // 16, sumrows, 0)

    pltpu.sync_copy(tsum, stage.at[s])
    plsc.subcore_barrier()

    my = pl.ds(s * _RPT, _RPT)
    pltpu.sync_copy(stage.at[0, my], rbuf)
    for r in range(1, _NT):
      pltpu.sync_copy(stage.at[r, my], tsum.at[pl.ds(0, _RPT)])

      def accrow(i, acc):
        sl = pl.ds(i * 16, 16)
        rbuf[sl] = rbuf[sl] + tsum[sl]
        return acc
      lax.fori_loop(0, _RPT // 16, accrow, 0)
    pltpu.sync_copy(rbuf, deg_o.at[c, my])

  mesh = plsc.VectorSubcoreMesh(**_MESH)
  call = pl.kernel(body, out_type=out, mesh=mesh, scratch_types=scratch,
                   compiler_params=pltpu.CompilerParams(
                       needs_layout_passes=False))
  return call(dst_i4)


def _row_spec(w):
  return pl.BlockSpec((_ROWB, w), lambda i: (i, 0))


def _full_spec(shape):
  return pl.BlockSpec(shape, lambda i: (0, 0))


def _tc_first(x, Ws, Wn, b):
  do = Ws.shape[1]
  half = do // 2

  def kbody(x_ref, ws_ref, wn_ref, b_ref, hs_ref, h0_ref, h1_ref):
    xb = x_ref[...]
    hs_ref[...] = jnp.dot(xb, ws_ref[...],
                          preferred_element_type=jnp.float32) + b_ref[...]
    hn = jnp.dot(xb, wn_ref[...], preferred_element_type=jnp.float32)
    h0_ref[...] = hn[:, :half]
    h1_ref[...] = hn[:, half:]

  return pl.pallas_call(
      kbody,
      grid=(_N // _ROWB,),
      in_specs=[_row_spec(x.shape[1]), _full_spec(Ws.shape),
                _full_spec(Wn.shape), _full_spec((1, do))],
      out_specs=[_row_spec(do), _row_spec(half), _row_spec(half)],
      out_shape=[jax.ShapeDtypeStruct((_N, do), jnp.float32),
                 jax.ShapeDtypeStruct((_N, half), jnp.float32),
                 jax.ShapeDtypeStruct((_N, half), jnp.float32)],
  )(x, Ws, Wn, b.reshape(1, -1))


def _tc_mid(hs_p, a0, a1, d0, d1, Ws, Wn, b, split):
  dp = hs_p.shape[1]
  hp = a0.shape[1]
  do = Ws.shape[1]
  half = do // 2

  def kbody(hs_ref, a0_ref, a1_ref, d0_ref, d1_ref, ws_ref, wn_ref, b_ref,
            hs_o, *hn_o):
    scale = 1.0 / jnp.maximum(d0_ref[...] + d1_ref[...], 1.0)
    h = hs_ref[...] + jnp.concatenate(
        [a0_ref[...] * scale, a1_ref[...] * scale], axis=1)
    h = jnp.maximum(h, 0.0)
    hs_o[...] = jnp.dot(h, ws_ref[...],
                        preferred_element_type=jnp.float32) + b_ref[...]
    hn = jnp.dot(h, wn_ref[...], preferred_element_type=jnp.float32)
    if split:
      hn_o[0][...] = hn[:, :half]
      hn_o[1][...] = hn[:, half:]
    else:
      hn_o[0][...] = hn

  if split:
    out_specs = [_row_spec(do), _row_spec(half), _row_spec(half)]
    out_shape = [jax.ShapeDtypeStruct((_N, do), jnp.float32),
                 jax.ShapeDtypeStruct((_N, half), jnp.float32),
                 jax.ShapeDtypeStruct((_N, half), jnp.float32)]
  else:
    out_specs = [_row_spec(do), _row_spec(do)]
    out_shape = [jax.ShapeDtypeStruct((_N, do), jnp.float32),
                 jax.ShapeDtypeStruct((_N, do), jnp.float32)]

  return pl.pallas_call(
      kbody,
      grid=(_N // _ROWB,),
      in_specs=[_row_spec(dp), _row_spec(hp), _row_spec(hp),
                pl.BlockSpec((_ROWB, 1), lambda i: (i, 0)),
                pl.BlockSpec((_ROWB, 1), lambda i: (i, 0)),
                _full_spec(Ws.shape), _full_spec(Wn.shape), _full_spec((1, do))],
      out_specs=out_specs,
      out_shape=out_shape,
  )(hs_p, a0, a1, d0, d1, Ws, Wn, b.reshape(1, -1))


def _tc_final(hs_p, p0, p1, d0, d1):
  dp = hs_p.shape[1]

  def kbody(hs_ref, p0_ref, p1_ref, d0_ref, d1_ref, out_ref):
    scale = 1.0 / jnp.maximum(d0_ref[...] + d1_ref[...], 1.0)
    out_ref[...] = hs_ref[...] + (p0_ref[...] + p1_ref[...]) * scale

  return pl.pallas_call(
      kbody,
      grid=(_N // _ROWB,),
      in_specs=[_row_spec(dp), _row_spec(dp), _row_spec(dp),
                pl.BlockSpec((_ROWB, 1), lambda i: (i, 0)),
                pl.BlockSpec((_ROWB, 1), lambda i: (i, 0))],
      out_specs=_row_spec(dp),
      out_shape=jax.ShapeDtypeStruct((_N, dp), jnp.float32),
  )(hs_p, p0, p1, d0, d1)


def kernel(x, edge_index, Ws1, Wn1, b1, Ws2, Wn2, b2, Ws3, Wn3, b3):
  src = edge_index[0]
  dst = edge_index[1]
  pad = _EPAD - _E
  pad_src = jnp.arange(pad, dtype=jnp.int32) % _N
  pad_dst = _N + (jnp.arange(pad, dtype=jnp.int32) % (_NPAD - _N))
  src_p = jnp.concatenate([src, pad_src])
  dst_p = jnp.concatenate([dst, pad_dst])
  src_i = src_p.reshape(_NT, _NCH, _CW)
  dst_i = dst_p.reshape(_NT, _NCH, _CW)
  src_i4 = src_p.reshape(2, _NT, _NCH2, _CW)
  dst_i4 = dst_p.reshape(2, _NT, _NCH2, _CW)
  degt = _deg_pass(dst_i4)
  d0 = degt[0, :_N, None]
  d1 = degt[1, :_N, None]

  hs1, hn10, hn11 = _tc_first(x, Ws1, Wn1, b1)
  agg10, agg11 = _edge_pass(hn10, hn11, src_i, dst_i, 128)

  hs2, hn20, hn21 = _tc_mid(hs1, agg10[:_N], agg11[:_N], d0, d1, Ws2, Wn2,
                            b2, split=True)
  agg20, agg21 = _edge_pass(hn20, hn21, src_i, dst_i, 128)

  hs3, hn3 = _tc_mid(hs2, agg20[:_N], agg21[:_N], d0, d1, Ws3, Wn3, b3,
                     split=False)
  p = _edge_pass_split(hn3, src_i4, dst_i4, 128)

  return _tc_final(hs3, p[0, :_N], p[1, :_N], d0, d1)


# speedup vs baseline: 7.7786x; 1.2150x over previous
import jax
import jax.numpy as jnp
from jax import lax
from jax.experimental import pallas as pl
from jax.experimental.pallas import tpu as pltpu
from jax.experimental.pallas import tpu_sc as plsc

_N = 10000
_E = 160000
_NPAD = 10240
_EPAD = 163840
_NT = 16
_NCH = 160
_NCH2 = 80
_CW = 64
_RPT = _NPAD // _NT
_DEGW = 16
_ROWB = 1000

_MESH = dict(core_axis_name="c", subcore_axis_name="s")


def _edge_pass(hn0, hn1, src_i, dst_i, W):
  outs = [jax.ShapeDtypeStruct((_NPAD, W), jnp.float32),
          jax.ShapeDtypeStruct((_NPAD, W), jnp.float32)]

  scratch = [
      pltpu.VMEM((_NCH2, _CW), jnp.int32),
      pltpu.VMEM((_NCH2, _CW), jnp.int32),
      pltpu.VMEM((_CW, W), jnp.float32),
      pltpu.VMEM((_CW, W), jnp.float32),
      pltpu.VMEM((16, W), jnp.float32),
      pltpu.VMEM_SHARED((_NPAD, W), jnp.float32),
      pltpu.SemaphoreType.DMA,
      pltpu.SemaphoreType.DMA,
  ]

  def body(hn0_ref, hn1_ref, src_ref, dst_ref, agg0_o, agg1_o,
           src_v, dst_v, rows0, rows1, zbuf, aggm, sem0, sem1):
    c = lax.axis_index("c")
    s = lax.axis_index("s")

    z16 = jnp.zeros((16,), jnp.float32)
    for r in range(16):
      for k in range(W // 16):
        zbuf[r, pl.ds(k * 16, 16)] = z16

    def zero_agg(i, acc):
      pltpu.sync_copy(zbuf, aggm.at[pl.ds(s * _RPT + i * 16, 16)])
      return acc
    lax.fori_loop(0, _RPT // 16, zero_agg, 0)

    plsc.subcore_barrier()

    last = _NCH2 // 2 - 1

    def run(hn_ref):
      def chunk_pair(j2, acc):
        a = j2 * 2
        pltpu.async_copy(hn_ref.at[src_v.at[a + 1]], rows1, sem1)
        pltpu.make_async_copy(hn_ref.at[src_v.at[a]], rows0, sem0).wait()
        pltpu.sync_copy(rows0, aggm.at[dst_v.at[a]], add=True)

        def prefetch():
          pltpu.async_copy(hn_ref.at[src_v.at[a + 2]], rows0, sem0)
        pl.when(j2 < last)(prefetch)
        pltpu.make_async_copy(hn_ref.at[src_v.at[a + 1]], rows1, sem1).wait()
        pltpu.sync_copy(rows1, aggm.at[dst_v.at[a + 1]], add=True)
        return acc
      for h in range(_NCH // _NCH2):
        pltpu.sync_copy(src_ref.at[s, pl.ds(h * _NCH2, _NCH2)], src_v)
        pltpu.sync_copy(dst_ref.at[s, pl.ds(h * _NCH2, _NCH2)], dst_v)
        pltpu.async_copy(hn_ref.at[src_v.at[0]], rows0, sem0)
        lax.fori_loop(0, _NCH2 // 2, chunk_pair, 0)

    pl.when(c == 0)(lambda: run(hn0_ref))
    pl.when(c == 1)(lambda: run(hn1_ref))
    plsc.subcore_barrier()

    rows_out = pl.ds(s * _RPT, _RPT)
    pl.when(c == 0)(lambda: pltpu.sync_copy(aggm.at[rows_out],
                                            agg0_o.at[rows_out]))
    pl.when(c == 1)(lambda: pltpu.sync_copy(aggm.at[rows_out],
                                            agg1_o.at[rows_out]))

  mesh = plsc.VectorSubcoreMesh(**_MESH)
  call = pl.kernel(body, out_type=outs, mesh=mesh, scratch_types=scratch)
  return call(hn0, hn1, src_i, dst_i)


def _edge_pass_split(hn, src_i4, dst_i4, W):
  outs = jax.ShapeDtypeStruct((2, _NPAD, W), jnp.float32)

  scratch = [
      pltpu.VMEM((_NCH2, _CW), jnp.int32),
      pltpu.VMEM((_NCH2, _CW), jnp.int32),
      pltpu.VMEM((_CW, W), jnp.float32),
      pltpu.VMEM((_CW, W), jnp.float32),
      pltpu.VMEM((16, W), jnp.float32),
      pltpu.VMEM_SHARED((_NPAD, W), jnp.float32),
      pltpu.SemaphoreType.DMA,
      pltpu.SemaphoreType.DMA,
  ]

  def body(hn_ref, src_ref, dst_ref, p_o,
           src_v, dst_v, rows0, rows1, zbuf, aggm, sem0, sem1):
    c = lax.axis_index("c")
    s = lax.axis_index("s")

    z16 = jnp.zeros((16,), jnp.float32)
    for r in range(16):
      for k in range(W // 16):
        zbuf[r, pl.ds(k * 16, 16)] = z16

    def zero_agg(i, acc):
      pltpu.sync_copy(zbuf, aggm.at[pl.ds(s * _RPT + i * 16, 16)])
      return acc
    lax.fori_loop(0, _RPT // 16, zero_agg, 0)

    pltpu.sync_copy(src_ref.at[c, s], src_v)
    pltpu.sync_copy(dst_ref.at[c, s], dst_v)
    plsc.subcore_barrier()

    last = _NCH2 // 2 - 1

    def chunk_pair(j2, acc):
      a = j2 * 2
      pltpu.async_copy(hn_ref.at[src_v.at[a + 1]], rows1, sem1)
      pltpu.make_async_copy(hn_ref.at[src_v.at[a]], rows0, sem0).wait()
      pltpu.sync_copy(rows0, aggm.at[dst_v.at[a]], add=True)

      def prefetch():
        pltpu.async_copy(hn_ref.at[src_v.at[a + 2]], rows0, sem0)
      pl.when(j2 < last)(prefetch)
      pltpu.make_async_copy(hn_ref.at[src_v.at[a + 1]], rows1, sem1).wait()
      pltpu.sync_copy(rows1, aggm.at[dst_v.at[a + 1]], add=True)
      return acc
    pltpu.async_copy(hn_ref.at[src_v.at[0]], rows0, sem0)
    lax.fori_loop(0, _NCH2 // 2, chunk_pair, 0)
    plsc.subcore_barrier()

    rows_out = pl.ds(s * _RPT, _RPT)
    pltpu.sync_copy(aggm.at[rows_out], p_o.at[c, rows_out])

  mesh = plsc.VectorSubcoreMesh(**_MESH)
  call = pl.kernel(body, out_type=outs, mesh=mesh, scratch_types=scratch)
  return call(hn, src_i4, dst_i4)


def _deg_pass(dst_i4):
  out = jax.ShapeDtypeStruct((2, _NPAD), jnp.float32)

  scratch = [
      pltpu.VMEM((_NCH2, _CW), jnp.int32),
      pltpu.VMEM((8 * _NPAD,), jnp.float32),
      pltpu.VMEM((_NPAD,), jnp.float32),
      pltpu.VMEM((_RPT,), jnp.float32),
      pltpu.VMEM_SHARED((_NT, _NPAD), jnp.float32),
  ]

  def body(dst_ref, deg_o, dst_v, hist, tsum, rbuf, stage):
    c = lax.axis_index("c")
    s = lax.axis_index("s")
    pltpu.sync_copy(dst_ref.at[c, s], dst_v)

    z16 = jnp.zeros((16,), jnp.float32)

    def zrow(i, acc):
      for r in range(8):
        hist[pl.ds(r * _NPAD + i * 16, 16)] = z16
      return acc
    lax.fori_loop(0, _NPAD // 16, zrow, 0)

    lanes = lax.iota(jnp.int32, 16)
    base = (lanes & 7) * _NPAD
    lo = lanes < 8
    hi = lanes >= 8
    ones = jnp.ones((16,), jnp.float32)

    def count(j, acc):
      for k in range(_CW // 16):
        d = dst_v[j, pl.ds(k * 16, 16)]
        plsc.addupdate_scatter(hist, [base + d], ones, mask=lo)
        plsc.addupdate_scatter(hist, [base + d], ones, mask=hi)
      return acc
    lax.fori_loop(0, _NCH2, count, 0)

    def sumrows(i, acc):
      t = hist[pl.ds(i * 16, 16)]
      for r in range(1, 8):
        t = t + hist[pl.ds(r * _NPAD + i * 16, 16)]
      tsum[pl.ds(i * 16, 16)] = t
      return acc
    lax.fori_loop(0, _NPAD // 16, sumrows, 0)

    pltpu.sync_copy(tsum, stage.at[s])
    plsc.subcore_barrier()

    my = pl.ds(s * _RPT, _RPT)
    pltpu.sync_copy(stage.at[0, my], rbuf)
    for r in range(1, _NT):
      pltpu.sync_copy(stage.at[r, my], tsum.at[pl.ds(0, _RPT)])

      def accrow(i, acc):
        sl = pl.ds(i * 16, 16)
        rbuf[sl] = rbuf[sl] + tsum[sl]
        return acc
      lax.fori_loop(0, _RPT // 16, accrow, 0)
    pltpu.sync_copy(rbuf, deg_o.at[c, my])

  mesh = plsc.VectorSubcoreMesh(**_MESH)
  call = pl.kernel(body, out_type=out, mesh=mesh, scratch_types=scratch,
                   compiler_params=pltpu.CompilerParams(
                       needs_layout_passes=False))
  return call(dst_i4)


def _row_spec(w):
  return pl.BlockSpec((_ROWB, w), lambda i: (i, 0))


def _full_spec(shape):
  return pl.BlockSpec(shape, lambda i: (0, 0))


def _tc_first(x, Ws, Wn, b):
  do = Ws.shape[1]
  half = do // 2

  def kbody(x_ref, ws_ref, wn_ref, b_ref, hs_ref, h0_ref, h1_ref):
    xb = x_ref[...]
    hs_ref[...] = jnp.dot(xb, ws_ref[...],
                          preferred_element_type=jnp.float32) + b_ref[...]
    hn = jnp.dot(xb, wn_ref[...], preferred_element_type=jnp.float32)
    h0_ref[...] = hn[:, :half]
    h1_ref[...] = hn[:, half:]

  return pl.pallas_call(
      kbody,
      grid=(_N // _ROWB,),
      in_specs=[_row_spec(x.shape[1]), _full_spec(Ws.shape),
                _full_spec(Wn.shape), _full_spec((1, do))],
      out_specs=[_row_spec(do), _row_spec(half), _row_spec(half)],
      out_shape=[jax.ShapeDtypeStruct((_N, do), jnp.float32),
                 jax.ShapeDtypeStruct((_N, half), jnp.float32),
                 jax.ShapeDtypeStruct((_N, half), jnp.float32)],
  )(x, Ws, Wn, b.reshape(1, -1))


def _tc_mid(hs_p, a0, a1, d0, d1, Ws, Wn, b, split):
  dp = hs_p.shape[1]
  hp = a0.shape[1]
  do = Ws.shape[1]
  half = do // 2

  def kbody(hs_ref, a0_ref, a1_ref, d0_ref, d1_ref, ws_ref, wn_ref, b_ref,
            hs_o, *hn_o):
    scale = 1.0 / jnp.maximum(d0_ref[...] + d1_ref[...], 1.0)
    h = hs_ref[...] + jnp.concatenate(
        [a0_ref[...] * scale, a1_ref[...] * scale], axis=1)
    h = jnp.maximum(h, 0.0)
    hs_o[...] = jnp.dot(h, ws_ref[...],
                        preferred_element_type=jnp.float32) + b_ref[...]
    hn = jnp.dot(h, wn_ref[...], preferred_element_type=jnp.float32)
    if split:
      hn_o[0][...] = hn[:, :half]
      hn_o[1][...] = hn[:, half:]
    else:
      hn_o[0][...] = hn

  if split:
    out_specs = [_row_spec(do), _row_spec(half), _row_spec(half)]
    out_shape = [jax.ShapeDtypeStruct((_N, do), jnp.float32),
                 jax.ShapeDtypeStruct((_N, half), jnp.float32),
                 jax.ShapeDtypeStruct((_N, half), jnp.float32)]
  else:
    out_specs = [_row_spec(do), _row_spec(do)]
    out_shape = [jax.ShapeDtypeStruct((_N, do), jnp.float32),
                 jax.ShapeDtypeStruct((_N, do), jnp.float32)]

  return pl.pallas_call(
      kbody,
      grid=(_N // _ROWB,),
      in_specs=[_row_spec(dp), _row_spec(hp), _row_spec(hp),
                pl.BlockSpec((_ROWB, 1), lambda i: (i, 0)),
                pl.BlockSpec((_ROWB, 1), lambda i: (i, 0)),
                _full_spec(Ws.shape), _full_spec(Wn.shape), _full_spec((1, do))],
      out_specs=out_specs,
      out_shape=out_shape,
  )(hs_p, a0, a1, d0, d1, Ws, Wn, b.reshape(1, -1))


def _tc_final(hs_p, p0, p1, d0, d1):
  dp = hs_p.shape[1]

  def kbody(hs_ref, p0_ref, p1_ref, d0_ref, d1_ref, out_ref):
    scale = 1.0 / jnp.maximum(d0_ref[...] + d1_ref[...], 1.0)
    out_ref[...] = hs_ref[...] + (p0_ref[...] + p1_ref[...]) * scale

  return pl.pallas_call(
      kbody,
      grid=(_N // _ROWB,),
      in_specs=[_row_spec(dp), _row_spec(dp), _row_spec(dp),
                pl.BlockSpec((_ROWB, 1), lambda i: (i, 0)),
                pl.BlockSpec((_ROWB, 1), lambda i: (i, 0))],
      out_specs=_row_spec(dp),
      out_shape=jax.ShapeDtypeStruct((_N, dp), jnp.float32),
  )(hs_p, p0, p1, d0, d1)


def kernel(x, edge_index, Ws1, Wn1, b1, Ws2, Wn2, b2, Ws3, Wn3, b3):
  src = edge_index[0]
  dst = edge_index[1]
  pad = _EPAD - _E
  pad_src = jnp.arange(pad, dtype=jnp.int32) % _N
  pad_dst = _N + (jnp.arange(pad, dtype=jnp.int32) % (_NPAD - _N))
  src_p = jnp.concatenate([src, pad_src])
  dst_p = jnp.concatenate([dst, pad_dst])
  src_i = src_p.reshape(_NT, _NCH, _CW)
  dst_i = dst_p.reshape(_NT, _NCH, _CW)
  src_i4 = src_p.reshape(2, _NT, _NCH2, _CW)
  dst_i4 = dst_p.reshape(2, _NT, _NCH2, _CW)
  degt = _deg_pass(dst_i4)
  d0 = degt[0, :_N, None]
  d1 = degt[1, :_N, None]

  hs1, hn10, hn11 = _tc_first(x, Ws1, Wn1, b1)
  agg10, agg11 = _edge_pass(hn10, hn11, src_i, dst_i, 128)

  hs2, hn20, hn21 = _tc_mid(hs1, agg10[:_N], agg11[:_N], d0, d1, Ws2, Wn2,
                            b2, split=True)
  agg20, agg21 = _edge_pass(hn20, hn21, src_i, dst_i, 128)

  hs3, hn3 = _tc_mid(hs2, agg20[:_N], agg21[:_N], d0, d1, Ws3, Wn3, b3,
                     split=False)
  p = _edge_pass_split(hn3, src_i4, dst_i4, 128)

  return _tc_final(hs3, p[0, :_N], p[1, :_N], d0, d1)
